# Initial kernel scaffold; baseline (speedup 1.0000x reference)
import jax
import jax.numpy as jnp
from jax.experimental import pallas as pl


def kernel(x, adj, w_in, b_in, wl1, bl1, wr1, br1, att1, cb1, wl2, bl2, wr2, br2, att2, cb2, g1, be1, g2, be2, wa, ba, wc, bc):
    raise NotImplementedError("write your pallas kernel here")


# speedup vs baseline: 11.2822x; 11.2822x over previous
import jax
import jax.numpy as jnp
from jax.experimental import pallas as pl
from jax.experimental.pallas import tpu as pltpu

_NEG = -1e30


def _acgnn_body(x_ref, adj_ref, win_ref, bin_ref,
                wlr1_ref, blr1t_ref, att1_ref, cb1_ref, g1_ref, be1_ref,
                wlr2_ref, blr2t_ref, att2_ref, cb2_ref, g2_ref, be2_ref,
                wac_ref, ba_ref, bc_ref, out_ref, *, heads, hd, n):
    hc = heads * hd

    x = x_ref[...]
    adjb = jnp.where(adj_ref[...] > 0.0, 0.0, _NEG)

    def ln_relu(v, gamma, beta, eps=1e-5):
        mu = jnp.mean(v, axis=-1, keepdims=True)
        d = v - mu
        var = jnp.mean(d * d, axis=-1, keepdims=True)
        return jnp.maximum(d * jax.lax.rsqrt(var + eps) * gamma + beta, 0.0)

    def gat(h, wlr_ref, blrt_ref, att_ref, cb_ref):
        gT = jax.lax.dot_general(wlr_ref[...], h, (((0,), (1,)), ((), ())),
                                 preferred_element_type=jnp.float32)
        gT = gT + blrt_ref[...]
        glT = gT[0:hc, :]
        grT = gT[hc:2 * hc, :]
        acc = None
        for head in range(heads):
            lo = head * hd
            a3 = att_ref[lo:lo + hd, :][:, :, None]
            s3 = grT[lo:lo + hd, :, None] + glT[lo:lo + hd, None, :]
            w3 = jnp.maximum(s3, 0.2 * s3)
            e = jnp.sum(w3 * a3, axis=0) + adjb
            e = e - jnp.max(e, axis=-1, keepdims=True)
            p = jnp.exp(e)
            p = p * pl.reciprocal(jnp.sum(p, axis=-1, keepdims=True),
                                  approx=True)
            ho = jax.lax.dot_general(p, glT[lo:lo + hd, :],
                                     (((1,), (1,)), ((), ())),
                                     preferred_element_type=jnp.float32)
            acc = ho if acc is None else acc + ho
        return acc * (1.0 / heads) + cb_ref[...]

    h0 = jnp.maximum(
        jnp.dot(x, win_ref[...], preferred_element_type=jnp.float32)
        + bin_ref[...], 0.0)
    h1 = ln_relu(h0 + gat(h0, wlr1_ref, blr1t_ref, att1_ref, cb1_ref),
                 g1_ref[...], be1_ref[...])
    h2 = ln_relu(h1 + gat(h1, wlr2_ref, blr2t_ref, att2_ref, cb2_ref),
                 g2_ref[...], be2_ref[...])

    out2 = jax.lax.dot_general(wac_ref[...], h2, (((1,), (1,)), ((), ())),
                               preferred_element_type=jnp.float32)
    logits = out2[0:1, :] + ba_ref[...]
    value = jnp.sum(out2[1:2, :], axis=-1, keepdims=True) * (1.0 / n) \
        + bc_ref[...]

    out_ref[...] = jnp.zeros((8, 128), jnp.float32)
    out_ref[0:1, :] = logits
    out_ref[1:2, :] = jnp.broadcast_to(value, (1, 128))


def kernel(x, adj, w_in, b_in, wl1, bl1, wr1, br1, att1, cb1,
           wl2, bl2, wr2, br2, att2, cb2, g1, be1, g2, be2,
           wa, ba, wc, bc):
    b, n, f = x.shape
    heads, hd = att1.shape
    hidden = w_in.shape[1]
    hc = heads * hd

    wlr1 = jnp.concatenate([wl1, wr1], axis=1)
    wlr2 = jnp.concatenate([wl2, wr2], axis=1)
    blr1t = jnp.concatenate([bl1, br1], axis=1).T
    blr2t = jnp.concatenate([bl2, br2], axis=1).T
    att1c = att1.reshape(hc, 1)
    att2c = att2.reshape(hc, 1)
    wac = jnp.concatenate(
        [wa.T, wc.T, jnp.zeros((6, hidden), jnp.float32)], axis=0)

    import functools
    body = functools.partial(_acgnn_body, heads=heads, hd=hd, n=n)

    def fixed(a):
        return pl.BlockSpec(a.shape, lambda i: (0,) * a.ndim)

    smalls = (w_in, b_in, wlr1, blr1t, att1c, cb1, g1, be1,
              wlr2, blr2t, att2c, cb2, g2, be2, wac, ba, bc)

    out = pl.pallas_call(
        body,
        grid=(b,),
        out_shape=jax.ShapeDtypeStruct((b, 8, 128), jnp.float32),
        in_specs=[
            pl.BlockSpec((None, n, f), lambda i: (i, 0, 0)),
            pl.BlockSpec((None, n, n), lambda i: (i, 0, 0)),
        ] + [fixed(a) for a in smalls],
        out_specs=pl.BlockSpec((None, 8, 128), lambda i: (i, 0, 0)),
        compiler_params=pltpu.CompilerParams(
            dimension_semantics=("parallel",)),
    )(x, adj, *smalls)

    return out[:, 0, :n], out[:, 1, 0]


# speedup vs baseline: 15.6833x; 1.3901x over previous
import functools

import jax
import jax.numpy as jnp
from jax.experimental import pallas as pl
from jax.experimental.pallas import tpu as pltpu

_NEG = -1e30


def _acgnn_body(x_ref, adj_ref, win_ref, bin_ref,
                wlr1_ref, blr1t_ref, asc1_ref, sgn1_ref, kap1_ref,
                cb1_ref, g1_ref, be1_ref,
                wlr2_ref, blr2t_ref, asc2_ref, sgn2_ref, kap2_ref,
                cb2_ref, g2_ref, be2_ref,
                wac_ref, ba_ref, bc_ref, out_ref, *, heads, hd, n):
    hc = heads * hd

    x = x_ref[...]
    adjb = jnp.where(adj_ref[...] > 0.0, 0.0, _NEG)

    def ln_relu(v, gamma, beta, eps=1e-5):
        mu = jnp.mean(v, axis=-1, keepdims=True)
        d = v - mu
        var = jnp.mean(d * d, axis=-1, keepdims=True)
        return jnp.maximum(d * jax.lax.rsqrt(var + eps) * gamma + beta, 0.0)

    def gat(h, wlr_ref, blrt_ref, asc_ref, sgn_ref, kap_ref, cb_ref):
        gT = jax.lax.dot_general(wlr_ref[...], h, (((0,), (1,)), ((), ())),
                                 preferred_element_type=jnp.float32)
        gT = gT + blrt_ref[...]
        glT = gT[0:hc, :]
        grT = gT[hc:2 * hc, :]
        asc = asc_ref[...]
        glTs = (glT * asc).astype(jnp.bfloat16)
        grTs = (grT * asc).astype(jnp.bfloat16)
        sgn = sgn_ref[...].astype(jnp.bfloat16)
        acc = None
        for head in range(heads):
            lo = head * hd
            lk = jax.lax.dot_general(kap_ref[lo:lo + hd, :],
                                     glT[lo:lo + hd, :],
                                     (((0,), (0,)), ((), ())),
                                     preferred_element_type=jnp.float32)
            u3 = grTs[lo:lo + hd, :, None] + glTs[lo:lo + hd, None, :]
            r3 = jnp.maximum(u3, 0) * sgn[lo:lo + hd, :][:, :, None]
            e = jnp.sum(r3, axis=0).astype(jnp.float32) + lk + adjb
            e = e - jnp.max(e, axis=-1, keepdims=True)
            p = jnp.exp(e)
            p = p * pl.reciprocal(jnp.sum(p, axis=-1, keepdims=True),
                                  approx=True)
            ho = jax.lax.dot_general(p, glT[lo:lo + hd, :],
                                     (((1,), (1,)), ((), ())),
                                     preferred_element_type=jnp.float32)
            acc = ho if acc is None else acc + ho
        return acc * (1.0 / heads) + cb_ref[...]

    h0 = jnp.maximum(
        jnp.dot(x, win_ref[...], preferred_element_type=jnp.float32)
        + bin_ref[...], 0.0)
    h1 = ln_relu(h0 + gat(h0, wlr1_ref, blr1t_ref, asc1_ref, sgn1_ref,
                          kap1_ref, cb1_ref),
                 g1_ref[...], be1_ref[...])
    h2 = ln_relu(h1 + gat(h1, wlr2_ref, blr2t_ref, asc2_ref, sgn2_ref,
                          kap2_ref, cb2_ref),
                 g2_ref[...], be2_ref[...])

    out2 = jax.lax.dot_general(wac_ref[...], h2, (((1,), (1,)), ((), ())),
                               preferred_element_type=jnp.float32)
    logits = out2[0:1, :] + ba_ref[...]
    value = jnp.sum(out2[1:2, :], axis=-1, keepdims=True) * (1.0 / n) \
        + bc_ref[...]

    out_ref[...] = jnp.zeros((8, 128), jnp.float32)
    out_ref[0:1, :] = logits
    out_ref[1:2, :] = jnp.broadcast_to(value, (1, 128))


def kernel(x, adj, w_in, b_in, wl1, bl1, wr1, br1, att1, cb1,
           wl2, bl2, wr2, br2, att2, cb2, g1, be1, g2, be2,
           wa, ba, wc, bc):
    b, n, f = x.shape
    heads, hd = att1.shape
    hidden = w_in.shape[1]
    hc = heads * hd

    wlr1 = jnp.concatenate([wl1, wr1], axis=1)
    wlr2 = jnp.concatenate([wl2, wr2], axis=1)
    blr1t = jnp.concatenate([bl1, br1], axis=1).T
    blr2t = jnp.concatenate([bl2, br2], axis=1).T

    def att_pack(att):
        a = att.reshape(hc, 1)
        asc = 0.8 * a
        sgn = jnp.where(a < 0, -1.0, 1.0).astype(jnp.float32)
        kap = 0.2 * a + 0.8 * jnp.where(a < 0, a, 0.0)
        return asc, sgn, kap

    asc1, sgn1, kap1 = att_pack(att1)
    asc2, sgn2, kap2 = att_pack(att2)
    wac = jnp.concatenate(
        [wa.T, wc.T, jnp.zeros((6, hidden), jnp.float32)], axis=0)

    body = functools.partial(_acgnn_body, heads=heads, hd=hd, n=n)

    def fixed(a):
        return pl.BlockSpec(a.shape, lambda i: (0,) * a.ndim)

    smalls = (w_in, b_in,
              wlr1, blr1t, asc1, sgn1, kap1, cb1, g1, be1,
              wlr2, blr2t, asc2, sgn2, kap2, cb2, g2, be2,
              wac, ba, bc)

    out = pl.pallas_call(
        body,
        grid=(b,),
        out_shape=jax.ShapeDtypeStruct((b, 8, 128), jnp.float32),
        in_specs=[
            pl.BlockSpec((None, n, f), lambda i: (i, 0, 0)),
            pl.BlockSpec((None, n, n), lambda i: (i, 0, 0)),
        ] + [fixed(a) for a in smalls],
        out_specs=pl.BlockSpec((None, 8, 128), lambda i: (i, 0, 0)),
        compiler_params=pltpu.CompilerParams(
            dimension_semantics=("parallel",)),
    )(x, adj, *smalls)

    return out[:, 0, :n], out[:, 1, 0]


# speedup vs baseline: 15.9586x; 1.0176x over previous
import functools

import jax
import jax.numpy as jnp
from jax.experimental import pallas as pl
from jax.experimental.pallas import tpu as pltpu

_NEG = -1e30


def _acgnn_body(x_ref, adj_ref, win_ref, bin_ref,
                wlr1_ref, blr1t_ref, asc1_ref, sgn1_ref, kap1_ref,
                cb1_ref, g1_ref, be1_ref,
                wlr2_ref, blr2t_ref, asc2_ref, sgn2_ref, kap2_ref,
                cb2_ref, g2_ref, be2_ref,
                wac_ref, ba_ref, bc_ref, out_ref, *, heads, hd, n):
    hc = heads * hd

    x = x_ref[...]
    adjb = jnp.where(adj_ref[...] > 0.0, 0.0, _NEG)

    def ln_relu(v, gamma, beta, eps=1e-5):
        mu = jnp.mean(v, axis=-1, keepdims=True)
        d = v - mu
        var = jnp.mean(d * d, axis=-1, keepdims=True)
        return jnp.maximum(d * jax.lax.rsqrt(var + eps) * gamma + beta, 0.0)

    def gat(h, wlr_ref, blrt_ref, asc_ref, sgn_ref, kap_ref, cb_ref):
        gT = jax.lax.dot_general(wlr_ref[...], h, (((0,), (1,)), ((), ())),
                                 preferred_element_type=jnp.float32)
        gT = gT + blrt_ref[...]
        glT = gT[0:hc, :]
        grT = gT[hc:2 * hc, :]
        asc = asc_ref[...]
        glTs = (glT * asc).astype(jnp.bfloat16)
        grTs = (grT * asc).astype(jnp.bfloat16)
        sgn = sgn_ref[...].astype(jnp.bfloat16)
        acc = None
        for head in range(heads):
            lo = head * hd
            lk = jax.lax.dot_general(kap_ref[lo:lo + hd, :],
                                     glT[lo:lo + hd, :],
                                     (((0,), (0,)), ((), ())),
                                     preferred_element_type=jnp.float32)
            u3 = grTs[lo:lo + hd, :, None] + glTs[lo:lo + hd, None, :]
            r3 = jnp.maximum(u3, 0) * sgn[lo:lo + hd, :][:, :, None]
            t = r3
            while t.shape[0] > 1:
                m = t.shape[0] // 2
                t = t[0:m] + t[m:2 * m]
            e = t[0].astype(jnp.float32) + lk + adjb
            e = e - jnp.max(e, axis=-1, keepdims=True)
            p = jnp.exp(e)
            p = p * pl.reciprocal(jnp.sum(p, axis=-1, keepdims=True),
                                  approx=True)
            ho = jax.lax.dot_general(p, glT[lo:lo + hd, :],
                                     (((1,), (1,)), ((), ())),
                                     preferred_element_type=jnp.float32)
            acc = ho if acc is None else acc + ho
        return acc * (1.0 / heads) + cb_ref[...]

    h0 = jnp.maximum(
        jnp.dot(x, win_ref[...], preferred_element_type=jnp.float32)
        + bin_ref[...], 0.0)
    h1 = ln_relu(h0 + gat(h0, wlr1_ref, blr1t_ref, asc1_ref, sgn1_ref,
                          kap1_ref, cb1_ref),
                 g1_ref[...], be1_ref[...])
    h2 = ln_relu(h1 + gat(h1, wlr2_ref, blr2t_ref, asc2_ref, sgn2_ref,
                          kap2_ref, cb2_ref),
                 g2_ref[...], be2_ref[...])

    out2 = jax.lax.dot_general(wac_ref[...], h2, (((1,), (1,)), ((), ())),
                               preferred_element_type=jnp.float32)
    logits = out2[0:1, :] + ba_ref[...]
    value = jnp.sum(out2[1:2, :], axis=-1, keepdims=True) * (1.0 / n) \
        + bc_ref[...]

    out_ref[...] = jnp.zeros((8, 128), jnp.float32)
    out_ref[0:1, :] = logits
    out_ref[1:2, :] = jnp.broadcast_to(value, (1, 128))


def kernel(x, adj, w_in, b_in, wl1, bl1, wr1, br1, att1, cb1,
           wl2, bl2, wr2, br2, att2, cb2, g1, be1, g2, be2,
           wa, ba, wc, bc):
    b, n, f = x.shape
    heads, hd = att1.shape
    hidden = w_in.shape[1]
    hc = heads * hd

    wlr1 = jnp.concatenate([wl1, wr1], axis=1)
    wlr2 = jnp.concatenate([wl2, wr2], axis=1)
    blr1t = jnp.concatenate([bl1, br1], axis=1).T
    blr2t = jnp.concatenate([bl2, br2], axis=1).T

    def att_pack(att):
        a = att.reshape(hc, 1)
        asc = 0.8 * a
        sgn = jnp.where(a < 0, -1.0, 1.0).astype(jnp.float32)
        kap = 0.2 * a + 0.8 * jnp.where(a < 0, a, 0.0)
        return asc, sgn, kap

    asc1, sgn1, kap1 = att_pack(att1)
    asc2, sgn2, kap2 = att_pack(att2)
    wac = jnp.concatenate(
        [wa.T, wc.T, jnp.zeros((6, hidden), jnp.float32)], axis=0)

    body = functools.partial(_acgnn_body, heads=heads, hd=hd, n=n)

    def fixed(a):
        return pl.BlockSpec(a.shape, lambda i: (0,) * a.ndim)

    smalls = (w_in, b_in,
              wlr1, blr1t, asc1, sgn1, kap1, cb1, g1, be1,
              wlr2, blr2t, asc2, sgn2, kap2, cb2, g2, be2,
              wac, ba, bc)

    out = pl.pallas_call(
        body,
        grid=(b,),
        out_shape=jax.ShapeDtypeStruct((b, 8, 128), jnp.float32),
        in_specs=[
            pl.BlockSpec((None, n, f), lambda i: (i, 0, 0)),
            pl.BlockSpec((None, n, n), lambda i: (i, 0, 0)),
        ] + [fixed(a) for a in smalls],
        out_specs=pl.BlockSpec((None, 8, 128), lambda i: (i, 0, 0)),
        compiler_params=pltpu.CompilerParams(
            dimension_semantics=("parallel",)),
    )(x, adj, *smalls)

    return out[:, 0, :n], out[:, 1, 0]


# speedup vs baseline: 16.2973x; 1.0212x over previous
import functools

import jax
import jax.numpy as jnp
from jax.experimental import pallas as pl
from jax.experimental.pallas import tpu as pltpu

_NEG = -1e30


def _acgnn_body(x_ref, adj_ref, win_ref, bin_ref,
                wlr1_ref, blr1t_ref, asc1_ref, sgn1_ref, kap1_ref,
                cb1_ref, g1_ref, be1_ref,
                wlr2_ref, blr2t_ref, asc2_ref, sgn2_ref, kap2_ref,
                cb2_ref, g2_ref, be2_ref,
                wac_ref, ba_ref, bc_ref, out_ref, *, heads, hd, n):
    hc = heads * hd

    x = x_ref[...]
    adjb = jnp.where(adj_ref[...] > 0.0, 0.0, _NEG)

    def ln_relu(v, gamma, beta, eps=1e-5):
        mu = jnp.mean(v, axis=-1, keepdims=True)
        d = v - mu
        var = jnp.mean(d * d, axis=-1, keepdims=True)
        return jnp.maximum(d * jax.lax.rsqrt(var + eps) * gamma + beta, 0.0)

    def gat(h, wlr_ref, blrt_ref, asc_ref, sgn_ref, kap_ref, cb_ref):
        gT = jax.lax.dot_general(wlr_ref[...], h, (((0,), (1,)), ((), ())),
                                 preferred_element_type=jnp.float32)
        gT = gT + blrt_ref[...]
        glT = gT[0:hc, :]
        grT = gT[hc:2 * hc, :]
        asc = asc_ref[...]
        glTs = (glT * asc).astype(jnp.bfloat16)
        grTs = (grT * asc).astype(jnp.bfloat16)
        sgn = sgn_ref[...].astype(jnp.bfloat16)
        acc = None
        for head in range(heads):
            lo = head * hd
            lk = jax.lax.dot_general(kap_ref[lo:lo + hd, :],
                                     glT[lo:lo + hd, :],
                                     (((0,), (0,)), ((), ())),
                                     preferred_element_type=jnp.float32)
            gr3 = grTs[lo:lo + hd, :, None]
            acc0 = acc1 = None
            for c in range(hd):
                sl = lo + c
                u3 = gr3[c:c + 1] + glTs[sl:sl + 1, None, :]
                r3 = jnp.maximum(u3, 0) * sgn[sl:sl + 1, :][:, :, None]
                if c % 2 == 0:
                    acc0 = r3 if acc0 is None else acc0 + r3
                else:
                    acc1 = r3 if acc1 is None else acc1 + r3
            e = (acc0 + acc1)[0].astype(jnp.float32) + lk + adjb
            e = e - jnp.max(e, axis=-1, keepdims=True)
            p = jnp.exp(e)
            p = p * pl.reciprocal(jnp.sum(p, axis=-1, keepdims=True),
                                  approx=True)
            ho = jax.lax.dot_general(p, glT[lo:lo + hd, :],
                                     (((1,), (1,)), ((), ())),
                                     preferred_element_type=jnp.float32)
            acc = ho if acc is None else acc + ho
        return acc * (1.0 / heads) + cb_ref[...]

    h0 = jnp.maximum(
        jnp.dot(x, win_ref[...], preferred_element_type=jnp.float32)
        + bin_ref[...], 0.0)
    h1 = ln_relu(h0 + gat(h0, wlr1_ref, blr1t_ref, asc1_ref, sgn1_ref,
                          kap1_ref, cb1_ref),
                 g1_ref[...], be1_ref[...])
    h2 = ln_relu(h1 + gat(h1, wlr2_ref, blr2t_ref, asc2_ref, sgn2_ref,
                          kap2_ref, cb2_ref),
                 g2_ref[...], be2_ref[...])

    out2 = jax.lax.dot_general(wac_ref[...], h2, (((1,), (1,)), ((), ())),
                               preferred_element_type=jnp.float32)
    logits = out2[0:1, :] + ba_ref[...]
    value = jnp.sum(out2[1:2, :], axis=-1, keepdims=True) * (1.0 / n) \
        + bc_ref[...]

    out_ref[...] = jnp.zeros((8, 128), jnp.float32)
    out_ref[0:1, :] = logits
    out_ref[1:2, :] = jnp.broadcast_to(value, (1, 128))


def kernel(x, adj, w_in, b_in, wl1, bl1, wr1, br1, att1, cb1,
           wl2, bl2, wr2, br2, att2, cb2, g1, be1, g2, be2,
           wa, ba, wc, bc):
    b, n, f = x.shape
    heads, hd = att1.shape
    hidden = w_in.shape[1]
    hc = heads * hd

    wlr1 = jnp.concatenate([wl1, wr1], axis=1)
    wlr2 = jnp.concatenate([wl2, wr2], axis=1)
    blr1t = jnp.concatenate([bl1, br1], axis=1).T
    blr2t = jnp.concatenate([bl2, br2], axis=1).T

    def att_pack(att):
        a = att.reshape(hc, 1)
        asc = 0.8 * a
        sgn = jnp.where(a < 0, -1.0, 1.0).astype(jnp.float32)
        kap = 0.2 * a + 0.8 * jnp.where(a < 0, a, 0.0)
        return asc, sgn, kap

    asc1, sgn1, kap1 = att_pack(att1)
    asc2, sgn2, kap2 = att_pack(att2)
    wac = jnp.concatenate(
        [wa.T, wc.T, jnp.zeros((6, hidden), jnp.float32)], axis=0)

    body = functools.partial(_acgnn_body, heads=heads, hd=hd, n=n)

    def fixed(a):
        return pl.BlockSpec(a.shape, lambda i: (0,) * a.ndim)

    smalls = (w_in, b_in,
              wlr1, blr1t, asc1, sgn1, kap1, cb1, g1, be1,
              wlr2, blr2t, asc2, sgn2, kap2, cb2, g2, be2,
              wac, ba, bc)

    out = pl.pallas_call(
        body,
        grid=(b,),
        out_shape=jax.ShapeDtypeStruct((b, 8, 128), jnp.float32),
        in_specs=[
            pl.BlockSpec((None, n, f), lambda i: (i, 0, 0)),
            pl.BlockSpec((None, n, n), lambda i: (i, 0, 0)),
        ] + [fixed(a) for a in smalls],
        out_specs=pl.BlockSpec((None, 8, 128), lambda i: (i, 0, 0)),
        compiler_params=pltpu.CompilerParams(
            dimension_semantics=("parallel",)),
    )(x, adj, *smalls)

    return out[:, 0, :n], out[:, 1, 0]


# speedup vs baseline: 18.3590x; 1.1265x over previous
import functools

import jax
import jax.numpy as jnp
from jax.experimental import pallas as pl
from jax.experimental.pallas import tpu as pltpu

_NEG = -1e30


def _acgnn_body(x_ref, adj_ref, win_ref, bin_ref,
                wlr1_ref, blr1t_ref, asc1_ref, sgn1_ref, kap1_ref,
                cb1_ref, g1_ref, be1_ref,
                wlr2_ref, blr2t_ref, asc2_ref, sgn2_ref, kap2_ref,
                cb2_ref, g2_ref, be2_ref,
                wac_ref, ba_ref, bc_ref, out_ref, *, heads, hd, n):
    hc = heads * hd

    x = x_ref[...]
    adjb = jnp.where(adj_ref[...] > 0.0, 0.0, _NEG)

    def ln_relu(v, gamma, beta, eps=1e-5):
        mu = jnp.mean(v, axis=-1, keepdims=True)
        d = v - mu
        var = jnp.mean(d * d, axis=-1, keepdims=True)
        return jnp.maximum(d * jax.lax.rsqrt(var + eps) * gamma + beta, 0.0)

    def gat(h, wlr_ref, blrt_ref, asc_ref, sgn_ref, kap_ref, cb_ref):
        gT = jax.lax.dot_general(wlr_ref[...], h, (((0,), (1,)), ((), ())),
                                 preferred_element_type=jnp.float32)
        gT = gT + blrt_ref[...]
        glT = gT[0:hc, :]
        grT = gT[hc:2 * hc, :]
        asc = asc_ref[...]
        glTs = (glT * asc).astype(jnp.bfloat16)
        grTs = (grT * asc).astype(jnp.bfloat16)
        sgn = sgn_ref[...].astype(jnp.bfloat16)
        es = []
        for head in range(heads):
            lo = head * hd
            lk = jax.lax.dot_general(kap_ref[lo:lo + hd, :],
                                     glT[lo:lo + hd, :],
                                     (((0,), (0,)), ((), ())),
                                     preferred_element_type=jnp.float32)
            gr3 = grTs[lo:lo + hd, :, None]
            acc0 = acc1 = None
            for c in range(hd):
                sl = lo + c
                u3 = gr3[c:c + 1] + glTs[sl:sl + 1, None, :]
                r3 = jnp.maximum(u3, 0) * sgn[sl:sl + 1, :][:, :, None]
                if c % 2 == 0:
                    acc0 = r3 if acc0 is None else acc0 + r3
                else:
                    acc1 = r3 if acc1 is None else acc1 + r3
            es.append((acc0 + acc1)[0].astype(jnp.float32) + lk + adjb)
        ee = jnp.concatenate(es, axis=0)
        ee = ee - jnp.max(ee, axis=-1, keepdims=True)
        p = jnp.exp(ee)
        p = p * pl.reciprocal(jnp.sum(p, axis=-1, keepdims=True),
                              approx=True)
        acc = None
        for head in range(heads):
            lo = head * hd
            ho = jax.lax.dot_general(p[head * n:(head + 1) * n, :],
                                     glT[lo:lo + hd, :],
                                     (((1,), (1,)), ((), ())),
                                     preferred_element_type=jnp.float32)
            acc = ho if acc is None else acc + ho
        return acc * (1.0 / heads) + cb_ref[...]

    h0 = jnp.maximum(
        jnp.dot(x, win_ref[...], preferred_element_type=jnp.float32)
        + bin_ref[...], 0.0)
    h1 = ln_relu(h0 + gat(h0, wlr1_ref, blr1t_ref, asc1_ref, sgn1_ref,
                          kap1_ref, cb1_ref),
                 g1_ref[...], be1_ref[...])
    h2 = ln_relu(h1 + gat(h1, wlr2_ref, blr2t_ref, asc2_ref, sgn2_ref,
                          kap2_ref, cb2_ref),
                 g2_ref[...], be2_ref[...])

    out2 = jax.lax.dot_general(wac_ref[...], h2, (((1,), (1,)), ((), ())),
                               preferred_element_type=jnp.float32)
    logits = out2[0:1, :] + ba_ref[...]
    value = jnp.sum(out2[1:2, :], axis=-1, keepdims=True) * (1.0 / n) \
        + bc_ref[...]

    out_ref[...] = jnp.zeros((8, 128), jnp.float32)
    out_ref[0:1, :] = logits
    out_ref[1:2, :] = jnp.broadcast_to(value, (1, 128))


def kernel(x, adj, w_in, b_in, wl1, bl1, wr1, br1, att1, cb1,
           wl2, bl2, wr2, br2, att2, cb2, g1, be1, g2, be2,
           wa, ba, wc, bc):
    b, n, f = x.shape
    heads, hd = att1.shape
    hidden = w_in.shape[1]
    hc = heads * hd

    wlr1 = jnp.concatenate([wl1, wr1], axis=1)
    wlr2 = jnp.concatenate([wl2, wr2], axis=1)
    blr1t = jnp.concatenate([bl1, br1], axis=1).T
    blr2t = jnp.concatenate([bl2, br2], axis=1).T

    def att_pack(att):
        a = att.reshape(hc, 1)
        asc = 0.8 * a
        sgn = jnp.where(a < 0, -1.0, 1.0).astype(jnp.float32)
        kap = 0.2 * a + 0.8 * jnp.where(a < 0, a, 0.0)
        return asc, sgn, kap

    asc1, sgn1, kap1 = att_pack(att1)
    asc2, sgn2, kap2 = att_pack(att2)
    wac = jnp.concatenate(
        [wa.T, wc.T, jnp.zeros((6, hidden), jnp.float32)], axis=0)

    body = functools.partial(_acgnn_body, heads=heads, hd=hd, n=n)

    def fixed(a):
        return pl.BlockSpec(a.shape, lambda i: (0,) * a.ndim)

    smalls = (w_in, b_in,
              wlr1, blr1t, asc1, sgn1, kap1, cb1, g1, be1,
              wlr2, blr2t, asc2, sgn2, kap2, cb2, g2, be2,
              wac, ba, bc)

    out = pl.pallas_call(
        body,
        grid=(b,),
        out_shape=jax.ShapeDtypeStruct((b, 8, 128), jnp.float32),
        in_specs=[
            pl.BlockSpec((None, n, f), lambda i: (i, 0, 0)),
            pl.BlockSpec((None, n, n), lambda i: (i, 0, 0)),
        ] + [fixed(a) for a in smalls],
        out_specs=pl.BlockSpec((None, 8, 128), lambda i: (i, 0, 0)),
        compiler_params=pltpu.CompilerParams(
            dimension_semantics=("parallel",)),
    )(x, adj, *smalls)

    return out[:, 0, :n], out[:, 1, 0]


# speedup vs baseline: 20.5617x; 1.1200x over previous
import functools

import jax
import jax.numpy as jnp
from jax.experimental import pallas as pl
from jax.experimental.pallas import tpu as pltpu

_NEG = -1e30


def _acgnn_body(x_ref, adj_ref, win_ref, bin_ref,
                wlr1_ref, blr1t_ref, asc1_ref, sgn1_ref, kap1_ref,
                cb1_ref, g1_ref, be1_ref,
                wlr2_ref, blr2t_ref, asc2_ref, sgn2_ref, kap2_ref,
                cb2_ref, g2_ref, be2_ref,
                wac_ref, ba_ref, bc_ref, out_ref, *, heads, hd, n):
    hc = heads * hd

    x = x_ref[...]
    adjb = jnp.where(adj_ref[...] > 0.0, 0.0, _NEG)

    def ln_relu(v, gamma, beta, eps=1e-5):
        mu = jnp.mean(v, axis=-1, keepdims=True)
        d = v - mu
        var = jnp.mean(d * d, axis=-1, keepdims=True)
        return jnp.maximum(d * jax.lax.rsqrt(var + eps) * gamma + beta, 0.0)

    def gat(h, wlr_ref, blrt_ref, asc_ref, sgn_ref, kap_ref, cb_ref):
        gT = jax.lax.dot_general(wlr_ref[...], h, (((0,), (1,)), ((), ())),
                                 preferred_element_type=jnp.float32)
        gT = gT + blrt_ref[...]
        glT = gT[0:hc, :]
        grT = gT[hc:2 * hc, :]
        asc = asc_ref[...]
        glTs = (glT * asc).astype(jnp.bfloat16)
        grTs = (grT * asc).astype(jnp.bfloat16)
        sgn = sgn_ref[...].astype(jnp.bfloat16)
        gr3 = grTs[:, :, None]
        accs = [None] * heads
        for c in range(hd):
            for head in range(heads):
                sl = head * hd + c
                u3 = gr3[sl:sl + 1] + glTs[sl:sl + 1, None, :]
                r3 = jnp.maximum(u3, 0) * sgn[sl:sl + 1, :][:, :, None]
                accs[head] = r3 if accs[head] is None else accs[head] + r3
        es = []
        for head in range(heads):
            lo = head * hd
            lk = jax.lax.dot_general(kap_ref[lo:lo + hd, :],
                                     glT[lo:lo + hd, :],
                                     (((0,), (0,)), ((), ())),
                                     preferred_element_type=jnp.float32)
            es.append(accs[head][0].astype(jnp.float32) + lk + adjb)
        ee = jnp.concatenate(es, axis=0)
        ee = ee - jnp.max(ee, axis=-1, keepdims=True)
        p = jnp.exp(ee)
        p = p * pl.reciprocal(jnp.sum(p, axis=-1, keepdims=True),
                              approx=True)
        acc = None
        for head in range(heads):
            lo = head * hd
            ho = jax.lax.dot_general(p[head * n:(head + 1) * n, :],
                                     glT[lo:lo + hd, :],
                                     (((1,), (1,)), ((), ())),
                                     preferred_element_type=jnp.float32)
            acc = ho if acc is None else acc + ho
        return acc * (1.0 / heads) + cb_ref[...]

    h0 = jnp.maximum(
        jnp.dot(x, win_ref[...], preferred_element_type=jnp.float32)
        + bin_ref[...], 0.0)
    h1 = ln_relu(h0 + gat(h0, wlr1_ref, blr1t_ref, asc1_ref, sgn1_ref,
                          kap1_ref, cb1_ref),
                 g1_ref[...], be1_ref[...])
    h2 = ln_relu(h1 + gat(h1, wlr2_ref, blr2t_ref, asc2_ref, sgn2_ref,
                          kap2_ref, cb2_ref),
                 g2_ref[...], be2_ref[...])

    out2 = jax.lax.dot_general(wac_ref[...], h2, (((1,), (1,)), ((), ())),
                               preferred_element_type=jnp.float32)
    logits = out2[0:1, :] + ba_ref[...]
    value = jnp.sum(out2[1:2, :], axis=-1, keepdims=True) * (1.0 / n) \
        + bc_ref[...]

    out_ref[...] = jnp.zeros((8, 128), jnp.float32)
    out_ref[0:1, :] = logits
    out_ref[1:2, :] = jnp.broadcast_to(value, (1, 128))


def kernel(x, adj, w_in, b_in, wl1, bl1, wr1, br1, att1, cb1,
           wl2, bl2, wr2, br2, att2, cb2, g1, be1, g2, be2,
           wa, ba, wc, bc):
    b, n, f = x.shape
    heads, hd = att1.shape
    hidden = w_in.shape[1]
    hc = heads * hd

    wlr1 = jnp.concatenate([wl1, wr1], axis=1)
    wlr2 = jnp.concatenate([wl2, wr2], axis=1)
    blr1t = jnp.concatenate([bl1, br1], axis=1).T
    blr2t = jnp.concatenate([bl2, br2], axis=1).T

    def att_pack(att):
        a = att.reshape(hc, 1)
        asc = 0.8 * a
        sgn = jnp.where(a < 0, -1.0, 1.0).astype(jnp.float32)
        kap = 0.2 * a + 0.8 * jnp.where(a < 0, a, 0.0)
        return asc, sgn, kap

    asc1, sgn1, kap1 = att_pack(att1)
    asc2, sgn2, kap2 = att_pack(att2)
    wac = jnp.concatenate(
        [wa.T, wc.T, jnp.zeros((6, hidden), jnp.float32)], axis=0)

    body = functools.partial(_acgnn_body, heads=heads, hd=hd, n=n)

    def fixed(a):
        return pl.BlockSpec(a.shape, lambda i: (0,) * a.ndim)

    smalls = (w_in, b_in,
              wlr1, blr1t, asc1, sgn1, kap1, cb1, g1, be1,
              wlr2, blr2t, asc2, sgn2, kap2, cb2, g2, be2,
              wac, ba, bc)

    out = pl.pallas_call(
        body,
        grid=(b,),
        out_shape=jax.ShapeDtypeStruct((b, 8, 128), jnp.float32),
        in_specs=[
            pl.BlockSpec((None, n, f), lambda i: (i, 0, 0)),
            pl.BlockSpec((None, n, n), lambda i: (i, 0, 0)),
        ] + [fixed(a) for a in smalls],
        out_specs=pl.BlockSpec((None, 8, 128), lambda i: (i, 0, 0)),
        compiler_params=pltpu.CompilerParams(
            dimension_semantics=("parallel",)),
    )(x, adj, *smalls)

    return out[:, 0, :n], out[:, 1, 0]
